# Initial kernel scaffold; baseline (speedup 1.0000x reference)
import jax
import jax.numpy as jnp
from jax.experimental import pallas as pl


def kernel(x, pos, edge_attr, params, edge_index):
    raise NotImplementedError("write your pallas kernel here")


# speedup vs baseline: 2.9677x; 2.9677x over previous
import functools

import jax
import jax.numpy as jnp
from jax import lax
from jax.experimental import pallas as pl
from jax.experimental.pallas import tpu as pltpu
from jax.experimental.pallas import tpu_sc as plsc

_F32 = jnp.float32
_NC, _NS = 2, 16
_NW = _NC * _NS
_CHUNK = 80


def _sc_mesh():
    return plsc.VectorSubcoreMesh(core_axis_name="c", subcore_axis_name="s")


@functools.cache
def _spmm_sc(n, e, d):
    epw = e // _NW
    steps = epw // _CHUNK
    assert epw * _NW == e and steps * _CHUNK == epw and d % 16 == 0
    nzchunks = n // _CHUNK
    assert nzchunks * _CHUNK == n
    per_tile = -(-nzchunks // _NS)

    @functools.partial(
        pl.kernel,
        out_type=jax.ShapeDtypeStruct((_NC, n, d), _F32),
        mesh=_sc_mesh(),
        scratch_types=[
            pltpu.VMEM((_CHUNK,), jnp.int32),
            pltpu.VMEM((_CHUNK,), jnp.int32),
            pltpu.VMEM((_CHUNK, 16), _F32),
            pltpu.VMEM((_CHUNK, d), _F32),
            pltpu.VMEM_SHARED((n, d), _F32),
            pltpu.SemaphoreType.DMA,
        ],
    )
    def spmm(m_hbm, row_hbm, col_hbm, nx_hbm, out_hbm, rowv, colv, nv, gbuf,
             acc, sem):
        c = lax.axis_index("c")
        s = lax.axis_index("s")
        wid = c * _NS + s
        zero16 = jnp.zeros((16,), _F32)

        def _zrow(r, _):
            for q in range(d // 16):
                gbuf[r, pl.ds(16 * q, 16)] = zero16
            return 0

        lax.fori_loop(0, _CHUNK, _zrow, 0)
        for i in range(per_tile):
            idx = s + _NS * i
            @pl.when(idx < nzchunks)
            def _():
                pltpu.sync_copy(gbuf, acc.at[pl.ds(idx * _CHUNK, _CHUNK)])
        plsc.subcore_barrier()

        ebase = wid * epw

        def step(j, _):
            off = ebase + j * _CHUNK
            pltpu.sync_copy(row_hbm.at[pl.ds(off, _CHUNK)], rowv)
            pltpu.sync_copy(col_hbm.at[pl.ds(off, _CHUNK)], colv)
            pltpu.sync_copy(nx_hbm.at[pl.ds(off, _CHUNK)], nv)
            pltpu.async_copy(m_hbm.at[rowv], gbuf, sem).wait()

            def srow(r, _):
                w = nv[r, :]
                for q in range(d // 16):
                    gbuf[r, pl.ds(16 * q, 16)] = gbuf[r, pl.ds(16 * q, 16)] * w
                return 0

            lax.fori_loop(0, _CHUNK, srow, 0)
            pltpu.sync_copy(gbuf, acc.at[colv], add=True)
            return 0

        lax.fori_loop(0, steps, step, 0)
        plsc.subcore_barrier()
        for i in range(per_tile):
            idx = s + _NS * i
            @pl.when(idx < nzchunks)
            def _():
                pltpu.sync_copy(acc.at[pl.ds(idx * _CHUNK, _CHUNK)],
                                out_hbm.at[c, pl.ds(idx * _CHUNK, _CHUNK)])

    return spmm


@functools.cache
def _nx_sc(n, e):
    epw = e // _NW
    steps = epw // _CHUNK
    assert epw * _NW == e and steps * _CHUNK == epw

    @functools.partial(
        pl.kernel,
        out_type=jax.ShapeDtypeStruct((e, 16), _F32),
        mesh=_sc_mesh(),
        scratch_types=[
            pltpu.VMEM((_CHUNK,), jnp.int32),
            pltpu.VMEM((_CHUNK,), jnp.int32),
            pltpu.VMEM((_CHUNK, 16), _F32),
            pltpu.VMEM((_CHUNK, 128), _F32),
            pltpu.VMEM((_CHUNK, 128), _F32),
            pltpu.VMEM((_CHUNK, 16), _F32),
            pltpu.SemaphoreType.DMA,
        ],
    )
    def normk(dis_hbm, row_hbm, col_hbm, ea_hbm, out_hbm, rowv, colv, eav,
              g1, g2, outv, sem):
        c = lax.axis_index("c")
        s = lax.axis_index("s")
        wid = c * _NS + s
        ebase = wid * epw

        def step(j, _):
            off = ebase + j * _CHUNK
            pltpu.sync_copy(row_hbm.at[pl.ds(off, _CHUNK)], rowv)
            pltpu.sync_copy(col_hbm.at[pl.ds(off, _CHUNK)], colv)
            pltpu.sync_copy(ea_hbm.at[pl.ds(off, _CHUNK)], eav)
            pltpu.async_copy(dis_hbm.at[rowv], g1, sem).wait()
            pltpu.async_copy(dis_hbm.at[colv], g2, sem).wait()

            def srow(r, _):
                outv[r, :] = g1[r, pl.ds(0, 16)] * g2[r, pl.ds(0, 16)] * eav[r, :]
                return 0

            lax.fori_loop(0, _CHUNK, srow, 0)
            pltpu.sync_copy(outv, out_hbm.at[pl.ds(off, _CHUNK)])
            return 0

        lax.fori_loop(0, steps, step, 0)

    return normk


def _gelu(h):
    return 0.5 * h * (1.0 + lax.erf(h * (2.0 ** -0.5)))


@functools.cache
def _add2_tc(n, d):
    def body(p_ref, o_ref):
        o_ref[...] = p_ref[0] + p_ref[1]

    return pl.pallas_call(body, out_shape=jax.ShapeDtypeStruct((n, d), _F32))


@functools.cache
def _dis_tc(n):
    def body(deg_ref, o_ref):
        deg = deg_ref[0] + deg_ref[1]
        o_ref[...] = jnp.where(deg > 0, lax.rsqrt(deg), 0.0)

    return pl.pallas_call(body, out_shape=jax.ShapeDtypeStruct((n, 128), _F32))


@functools.cache
def _mlpup_tc(n, ins, emb):
    def body(x_ref, w1_ref, b1_ref, w2_ref, b2_ref, o_ref):
        h = jnp.dot(x_ref[...], w1_ref[...], preferred_element_type=_F32)
        h = _gelu(h + b1_ref[...])
        o_ref[...] = (jnp.dot(h, w2_ref[...], preferred_element_type=_F32)
                      + b2_ref[...])

    return pl.pallas_call(body, out_shape=jax.ShapeDtypeStruct((n, emb), _F32))


@functools.cache
def _combine_tc(n, dp, do, act):
    def body(m0_ref, m1_ref, m2_ref, p3_ref, w_ref, b_ref, o_ref):
        acc = jnp.dot(m0_ref[...], w_ref[0], preferred_element_type=_F32)
        acc += jnp.dot(m1_ref[...], w_ref[1], preferred_element_type=_F32)
        acc += jnp.dot(m2_ref[...], w_ref[2], preferred_element_type=_F32)
        acc += jnp.dot(p3_ref[0] + p3_ref[1], w_ref[3],
                       preferred_element_type=_F32)
        acc += b_ref[...]
        if act == "relu":
            o_ref[...] = jnp.maximum(acc, 0.0)
        else:
            shifted = acc - jnp.max(acc, axis=1, keepdims=True)
            lse = jnp.log(jnp.sum(jnp.exp(shifted), axis=1, keepdims=True))
            o_ref[...] = shifted - lse

    return pl.pallas_call(body, out_shape=jax.ShapeDtypeStruct((n, do), _F32))


@functools.cache
def _mlpdown_tc(n, emb):
    def body(z_ref, w1_ref, b1_ref, w2_ref, b2_ref, o_ref):
        h = jnp.dot(z_ref[...], w1_ref[...], preferred_element_type=_F32)
        g = _gelu(h + b1_ref[...])
        o_ref[...] = g * w2_ref[0, 0] + b2_ref[...]

    return pl.pallas_call(body, out_shape=jax.ShapeDtypeStruct((n, 1), _F32))


def _hop(m, row, col, nx16, n, e):
    d = m.shape[1]
    part = _spmm_sc(n, e, d)(m, row, col, nx16)
    return _add2_tc(n, d)(part)


def kernel(x, pos, edge_attr, params, edge_index):
    n, e = x.shape[0], edge_attr.shape[0]
    row, col = edge_index[0], edge_index[1]

    ea16 = jnp.broadcast_to(edge_attr[:, None], (e, 16))
    ones128 = jnp.ones((n, 128), _F32)
    deg2 = _spmm_sc(n, e, 128)(ones128, row, col, ea16)
    dis128 = _dis_tc(n)(deg2)
    nx16 = _nx_sc(n, e)(dis128, row, col, ea16)

    pu = params["mlpup"]
    z = _mlpup_tc(n, x.shape[1], pu["w1"].shape[1])(
        x, pu["w1"], pu["b1"], pu["w2"], pu["b2"])

    emb = z.shape[1]
    dp1 = 128
    w1p = jnp.pad(params["conv1"]["w"],
                  ((0, 0), (0, dp1 - params["conv1"]["w"].shape[1]), (0, 0)))
    pad = jnp.zeros((n, dp1 - emb - pos.shape[1]), _F32)

    def tag_layer(m0, w, b, act):
        d = m0.shape[1]
        m1 = _hop(m0, row, col, nx16, n, e)
        m2 = _hop(m1, row, col, nx16, n, e)
        p3 = _spmm_sc(n, e, d)(m2, row, col, nx16)
        return _combine_tc(n, d, w.shape[2], act)(m0, m1, m2, p3, w, b)

    for _ in range(4):
        h0 = jnp.concatenate([z, pos, pad], axis=1)
        h1 = tag_layer(h0, w1p, params["conv1"]["b"], "relu")
        h2 = tag_layer(h1, params["conv2"]["w"], params["conv2"]["b"], "relu")
        z = tag_layer(h2, params["conv3"]["w"], params["conv3"]["b"], "lsm")

    pd = params["mlpdown"]
    out = _mlpdown_tc(n, emb)(z, pd["w1"], pd["b1"], pd["w2"], pd["b2"])
    return (out, z)


# speedup vs baseline: 7.3425x; 2.4741x over previous
import functools

import jax
import jax.numpy as jnp
from jax import lax
from jax.experimental import pallas as pl
from jax.experimental.pallas import tpu as pltpu
from jax.experimental.pallas import tpu_sc as plsc

_F32 = jnp.float32
_NC, _NS = 2, 16
_NW = _NC * _NS
_CHUNK = 80


def _sc_mesh():
    return plsc.VectorSubcoreMesh(core_axis_name="c", subcore_axis_name="s")


_ISTEPS = 25


@functools.cache
def _spmm_sc(n, e, d):
    epw = e // _NW
    steps = epw // _CHUNK
    nsup = steps // _ISTEPS
    assert epw * _NW == e and steps * _CHUNK == epw and d % 16 == 0
    assert nsup * _ISTEPS == steps
    nzchunks = n // _CHUNK
    assert nzchunks * _CHUNK == n
    per_tile = -(-nzchunks // _NS)

    @functools.partial(
        pl.kernel,
        out_type=jax.ShapeDtypeStruct((_NC, n, d), _F32),
        mesh=_sc_mesh(),
        scratch_types=[
            pltpu.VMEM((_ISTEPS, 1, _CHUNK), jnp.int32),
            pltpu.VMEM((_ISTEPS, 1, _CHUNK), jnp.int32),
            pltpu.VMEM((_CHUNK // 8, 128), _F32),
            pltpu.VMEM((_CHUNK // 8, 128), _F32),
            pltpu.VMEM((_CHUNK, d), _F32),
            pltpu.VMEM((_CHUNK, d), _F32),
            pltpu.VMEM_SHARED((n, d), _F32),
            pltpu.SemaphoreType.DMA,
            pltpu.SemaphoreType.DMA,
            pltpu.SemaphoreType.DMA,
            pltpu.SemaphoreType.DMA,
            pltpu.SemaphoreType.DMA,
            pltpu.SemaphoreType.DMA,
        ],
    )
    def spmm(m_hbm, row_hbm, col_hbm, nx_hbm, out_hbm, rsup, csup, nv0, nv1,
             gb0, gb1, acc, gsem0, gsem1, ssem0, ssem1, nsem0, nsem1):
        c = lax.axis_index("c")
        s = lax.axis_index("s")
        wid = c * _NS + s

        def wait_gather(sem):
            pltpu.make_async_copy(m_hbm.at[rsup.at[0, 0]], gb0, sem).wait()

        def wait_scatter(sem):
            pltpu.make_async_copy(gb0, acc.at[csup.at[0, 0]], sem).wait()

        def wait_nv(sem):
            pltpu.make_async_copy(nx_hbm.at[0], nv0, sem).wait()

        zero16 = jnp.zeros((16,), _F32)

        def _zrow(r, _):
            for q in range(d // 16):
                gb0[r, pl.ds(16 * q, 16)] = zero16
            return 0

        lax.fori_loop(0, _CHUNK, _zrow, 0)
        for i in range(per_tile):
            idx = s + _NS * i
            @pl.when(idx < nzchunks)
            def _():
                pltpu.sync_copy(gb0, acc.at[pl.ds(idx * _CHUNK, _CHUNK)])
        plsc.subcore_barrier()

        sbase = wid * steps

        def scale(gb, nv):
            def srow(r, _):
                w = nv[r // 8, pl.ds(16 * (r % 8), 16)]
                for q in range(d // 16):
                    gb[r, pl.ds(16 * q, 16)] = gb[r, pl.ds(16 * q, 16)] * w
                return 0
            lax.fori_loop(0, _CHUNK, srow, 0, unroll=4)

        for sup in range(nsup):
            off = sbase + sup * _ISTEPS
            pltpu.sync_copy(row_hbm.at[pl.ds(off, _ISTEPS)], rsup)
            pltpu.sync_copy(col_hbm.at[pl.ds(off, _ISTEPS)], csup)
            pltpu.async_copy(m_hbm.at[rsup.at[0, 0]], gb0, gsem0)
            pltpu.async_copy(nx_hbm.at[off], nv0, nsem0)

            def dstep(j2, _):
                j = 2 * j2
                @pl.when(j2 > 0)
                def _():
                    wait_scatter(ssem1)
                pltpu.async_copy(m_hbm.at[rsup.at[j + 1, 0]], gb1, gsem1)
                pltpu.async_copy(nx_hbm.at[off + j + 1], nv1, nsem1)
                wait_gather(gsem0)
                wait_nv(nsem0)
                scale(gb0, nv0)
                pltpu.async_copy(gb0, acc.at[csup.at[j, 0]], ssem0, add=True)
                wait_scatter(ssem0)
                pltpu.async_copy(m_hbm.at[rsup.at[j + 2, 0]], gb0, gsem0)
                pltpu.async_copy(nx_hbm.at[off + j + 2], nv0, nsem0)
                wait_gather(gsem1)
                wait_nv(nsem1)
                scale(gb1, nv1)
                pltpu.async_copy(gb1, acc.at[csup.at[j + 1, 0]], ssem1, add=True)
                return 0

            lax.fori_loop(0, (_ISTEPS - 1) // 2, dstep, 0)
            wait_gather(gsem0)
            wait_nv(nsem0)
            scale(gb0, nv0)
            pltpu.async_copy(gb0, acc.at[csup.at[_ISTEPS - 1, 0]], ssem0, add=True)
            wait_scatter(ssem1)
            wait_scatter(ssem0)

        plsc.subcore_barrier()
        for i in range(per_tile):
            idx = s + _NS * i
            @pl.when(idx < nzchunks)
            def _():
                pltpu.sync_copy(acc.at[pl.ds(idx * _CHUNK, _CHUNK)],
                                out_hbm.at[c, pl.ds(idx * _CHUNK, _CHUNK)])

    return spmm


@functools.cache
def _nx_sc(n, e):
    epw = e // _NW
    steps = epw // _CHUNK
    assert epw * _NW == e and steps * _CHUNK == epw

    @functools.partial(
        pl.kernel,
        out_type=jax.ShapeDtypeStruct((e, 16), _F32),
        mesh=_sc_mesh(),
        scratch_types=[
            pltpu.VMEM((_CHUNK,), jnp.int32),
            pltpu.VMEM((_CHUNK,), jnp.int32),
            pltpu.VMEM((_CHUNK, 16), _F32),
            pltpu.VMEM((_CHUNK, 128), _F32),
            pltpu.VMEM((_CHUNK, 128), _F32),
            pltpu.VMEM((_CHUNK, 16), _F32),
            pltpu.SemaphoreType.DMA,
        ],
    )
    def normk(dis_hbm, row_hbm, col_hbm, ea_hbm, out_hbm, rowv, colv, eav,
              g1, g2, outv, sem):
        c = lax.axis_index("c")
        s = lax.axis_index("s")
        wid = c * _NS + s
        ebase = wid * epw

        def step(j, _):
            off = ebase + j * _CHUNK
            pltpu.sync_copy(row_hbm.at[pl.ds(off, _CHUNK)], rowv)
            pltpu.sync_copy(col_hbm.at[pl.ds(off, _CHUNK)], colv)
            pltpu.sync_copy(ea_hbm.at[pl.ds(off, _CHUNK)], eav)
            pltpu.async_copy(dis_hbm.at[rowv], g1, sem).wait()
            pltpu.async_copy(dis_hbm.at[colv], g2, sem).wait()

            def srow(r, _):
                outv[r, :] = g1[r, pl.ds(0, 16)] * g2[r, pl.ds(0, 16)] * eav[r, :]
                return 0

            lax.fori_loop(0, _CHUNK, srow, 0)
            pltpu.sync_copy(outv, out_hbm.at[pl.ds(off, _CHUNK)])
            return 0

        lax.fori_loop(0, steps, step, 0)

    return normk


def _gelu(h):
    return 0.5 * h * (1.0 + lax.erf(h * (2.0 ** -0.5)))


@functools.cache
def _add2_tc(n, d):
    def body(p_ref, o_ref):
        o_ref[...] = p_ref[0] + p_ref[1]

    return pl.pallas_call(body, out_shape=jax.ShapeDtypeStruct((n, d), _F32))


@functools.cache
def _dis_tc(n):
    def body(deg_ref, o_ref):
        deg = deg_ref[0] + deg_ref[1]
        o_ref[...] = jnp.where(deg > 0, lax.rsqrt(deg), 0.0)

    return pl.pallas_call(body, out_shape=jax.ShapeDtypeStruct((n, 128), _F32))


@functools.cache
def _mlpup_tc(n, ins, emb):
    def body(x_ref, w1_ref, b1_ref, w2_ref, b2_ref, o_ref):
        h = jnp.dot(x_ref[...], w1_ref[...], preferred_element_type=_F32)
        h = _gelu(h + b1_ref[...])
        o_ref[...] = (jnp.dot(h, w2_ref[...], preferred_element_type=_F32)
                      + b2_ref[...])

    return pl.pallas_call(body, out_shape=jax.ShapeDtypeStruct((n, emb), _F32))


@functools.cache
def _combine_tc(n, dp, do, act):
    def body(m0_ref, m1_ref, m2_ref, p3_ref, w_ref, b_ref, o_ref):
        acc = jnp.dot(m0_ref[...], w_ref[0], preferred_element_type=_F32)
        acc += jnp.dot(m1_ref[...], w_ref[1], preferred_element_type=_F32)
        acc += jnp.dot(m2_ref[...], w_ref[2], preferred_element_type=_F32)
        acc += jnp.dot(p3_ref[0] + p3_ref[1], w_ref[3],
                       preferred_element_type=_F32)
        acc += b_ref[...]
        if act == "relu":
            o_ref[...] = jnp.maximum(acc, 0.0)
        else:
            shifted = acc - jnp.max(acc, axis=1, keepdims=True)
            lse = jnp.log(jnp.sum(jnp.exp(shifted), axis=1, keepdims=True))
            o_ref[...] = shifted - lse

    return pl.pallas_call(body, out_shape=jax.ShapeDtypeStruct((n, do), _F32))


@functools.cache
def _mlpdown_tc(n, emb):
    def body(z_ref, w1_ref, b1_ref, w2_ref, b2_ref, o_ref):
        h = jnp.dot(z_ref[...], w1_ref[...], preferred_element_type=_F32)
        g = _gelu(h + b1_ref[...])
        o_ref[...] = g * w2_ref[0, 0] + b2_ref[...]

    return pl.pallas_call(body, out_shape=jax.ShapeDtypeStruct((n, 1), _F32))


def _hop(m, row2, col2, nx3, n, e):
    d = m.shape[1]
    part = _spmm_sc(n, e, d)(m, row2, col2, nx3)
    return _add2_tc(n, d)(part)


def kernel(x, pos, edge_attr, params, edge_index):
    n, e = x.shape[0], edge_attr.shape[0]
    row, col = edge_index[0], edge_index[1]
    row2 = row.reshape(-1, 1, _CHUNK)
    col2 = col.reshape(-1, 1, _CHUNK)

    ea16 = jnp.broadcast_to(edge_attr[:, None], (e, 16))
    ea3 = ea16.reshape(-1, _CHUNK // 8, 128)
    ones128 = jnp.ones((n, 128), _F32)
    deg2 = _spmm_sc(n, e, 128)(ones128, row2, col2, ea3)
    dis128 = _dis_tc(n)(deg2)
    nx3 = _nx_sc(n, e)(dis128, row, col, ea16).reshape(-1, _CHUNK // 8, 128)

    pu = params["mlpup"]
    z = _mlpup_tc(n, x.shape[1], pu["w1"].shape[1])(
        x, pu["w1"], pu["b1"], pu["w2"], pu["b2"])

    emb = z.shape[1]
    dp1 = 128
    w1p = jnp.pad(params["conv1"]["w"],
                  ((0, 0), (0, dp1 - params["conv1"]["w"].shape[1]), (0, 0)))
    pad = jnp.zeros((n, dp1 - emb - pos.shape[1]), _F32)

    def tag_layer(m0, w, b, act):
        d = m0.shape[1]
        m1 = _hop(m0, row2, col2, nx3, n, e)
        m2 = _hop(m1, row2, col2, nx3, n, e)
        p3 = _spmm_sc(n, e, d)(m2, row2, col2, nx3)
        return _combine_tc(n, d, w.shape[2], act)(m0, m1, m2, p3, w, b)

    for _ in range(4):
        h0 = jnp.concatenate([z, pos, pad], axis=1)
        h1 = tag_layer(h0, w1p, params["conv1"]["b"], "relu")
        h2 = tag_layer(h1, params["conv2"]["w"], params["conv2"]["b"], "relu")
        z = tag_layer(h2, params["conv3"]["w"], params["conv3"]["b"], "lsm")

    pd = params["mlpdown"]
    out = _mlpdown_tc(n, emb)(z, pd["w1"], pd["b1"], pd["w2"], pd["b2"])
    return (out, z)


# speedup vs baseline: 8.0607x; 1.0978x over previous
import functools

import jax
import jax.numpy as jnp
from jax import lax
from jax.experimental import pallas as pl
from jax.experimental.pallas import tpu as pltpu
from jax.experimental.pallas import tpu_sc as plsc

_F32 = jnp.float32
_NC, _NS = 2, 16
_NW = _NC * _NS
_CHUNK = 80


def _sc_mesh():
    return plsc.VectorSubcoreMesh(core_axis_name="c", subcore_axis_name="s")


_ISTEPS = 25


@functools.cache
def _spmm_sc(n, e, d):
    epw = e // _NW
    steps = epw // _CHUNK
    nsup = steps // _ISTEPS
    assert epw * _NW == e and steps * _CHUNK == epw and d % 16 == 0
    assert nsup * _ISTEPS == steps
    nzchunks = n // _CHUNK
    assert nzchunks * _CHUNK == n
    per_tile = -(-nzchunks // _NS)

    @functools.partial(
        pl.kernel,
        out_type=jax.ShapeDtypeStruct((_NC, n, d), _F32),
        mesh=_sc_mesh(),
        scratch_types=[
            pltpu.VMEM((_ISTEPS, 1, _CHUNK), jnp.int32),
            pltpu.VMEM((_ISTEPS, 1, _CHUNK), jnp.int32),
            pltpu.VMEM((_CHUNK // 8, 128), _F32),
            pltpu.VMEM((_CHUNK // 8, 128), _F32),
            pltpu.VMEM((_CHUNK, d), _F32),
            pltpu.VMEM((_CHUNK, d), _F32),
            pltpu.VMEM_SHARED((n, d), _F32),
            pltpu.SemaphoreType.DMA,
            pltpu.SemaphoreType.DMA,
            pltpu.SemaphoreType.DMA,
            pltpu.SemaphoreType.DMA,
            pltpu.SemaphoreType.DMA,
            pltpu.SemaphoreType.DMA,
        ],
    )
    def spmm(m_hbm, row_hbm, col_hbm, nx_hbm, out_hbm, rsup, csup, nv0, nv1,
             gb0, gb1, acc, gsem0, gsem1, ssem0, ssem1, nsem0, nsem1):
        c = lax.axis_index("c")
        s = lax.axis_index("s")
        wid = c * _NS + s

        def wait_gather(sem):
            pltpu.make_async_copy(m_hbm.at[rsup.at[0, 0]], gb0, sem).wait()

        def wait_scatter(sem):
            pltpu.make_async_copy(gb0, acc.at[csup.at[0, 0]], sem).wait()

        def wait_nv(sem):
            pltpu.make_async_copy(nx_hbm.at[0], nv0, sem).wait()

        zero16 = jnp.zeros((16,), _F32)

        def _zrow(r, _):
            for q in range(d // 16):
                gb0[r, pl.ds(16 * q, 16)] = zero16
            return 0

        lax.fori_loop(0, _CHUNK, _zrow, 0)
        for i in range(per_tile):
            idx = s + _NS * i
            @pl.when(idx < nzchunks)
            def _():
                pltpu.sync_copy(gb0, acc.at[pl.ds(idx * _CHUNK, _CHUNK)])
        plsc.subcore_barrier()

        sbase = wid * steps

        def scale(gb, nv):
            def srow(r, _):
                w = nv[r // 8, pl.ds(16 * (r % 8), 16)]
                for q in range(d // 16):
                    gb[r, pl.ds(16 * q, 16)] = gb[r, pl.ds(16 * q, 16)] * w
                return 0
            lax.fori_loop(0, _CHUNK, srow, 0, unroll=4)

        for sup in range(nsup):
            off = sbase + sup * _ISTEPS
            pltpu.sync_copy(row_hbm.at[pl.ds(off, _ISTEPS)], rsup)
            pltpu.sync_copy(col_hbm.at[pl.ds(off, _ISTEPS)], csup)
            pltpu.async_copy(m_hbm.at[rsup.at[0, 0]], gb0, gsem0)
            pltpu.async_copy(nx_hbm.at[off], nv0, nsem0)

            def dstep(j2, _):
                j = 2 * j2
                @pl.when(j2 > 0)
                def _():
                    wait_scatter(ssem1)
                pltpu.async_copy(m_hbm.at[rsup.at[j + 1, 0]], gb1, gsem1)
                pltpu.async_copy(nx_hbm.at[off + j + 1], nv1, nsem1)
                wait_gather(gsem0)
                wait_nv(nsem0)
                scale(gb0, nv0)
                pltpu.async_copy(gb0, acc.at[csup.at[j, 0]], ssem0, add=True)
                wait_scatter(ssem0)
                pltpu.async_copy(m_hbm.at[rsup.at[j + 2, 0]], gb0, gsem0)
                pltpu.async_copy(nx_hbm.at[off + j + 2], nv0, nsem0)
                wait_gather(gsem1)
                wait_nv(nsem1)
                scale(gb1, nv1)
                pltpu.async_copy(gb1, acc.at[csup.at[j + 1, 0]], ssem1, add=True)
                return 0

            lax.fori_loop(0, (_ISTEPS - 1) // 2, dstep, 0)
            wait_gather(gsem0)
            wait_nv(nsem0)
            scale(gb0, nv0)
            pltpu.async_copy(gb0, acc.at[csup.at[_ISTEPS - 1, 0]], ssem0, add=True)
            wait_scatter(ssem1)
            wait_scatter(ssem0)

        plsc.subcore_barrier()
        for i in range(per_tile):
            idx = s + _NS * i
            @pl.when(idx < nzchunks)
            def _():
                pltpu.sync_copy(acc.at[pl.ds(idx * _CHUNK, _CHUNK)],
                                out_hbm.at[c, pl.ds(idx * _CHUNK, _CHUNK)])

    return spmm


@functools.cache
def _nx_sc(n, e):
    epw = e // _NW
    steps = epw // _CHUNK
    assert epw * _NW == e and steps * _CHUNK == epw

    nsup = steps // _ISTEPS
    assert nsup * _ISTEPS == steps
    g_shape = (_CHUNK, 128)
    v_shape = (_CHUNK // 8, 128)

    @functools.partial(
        pl.kernel,
        out_type=jax.ShapeDtypeStruct((e // _CHUNK, _CHUNK // 8, 128), _F32),
        mesh=_sc_mesh(),
        scratch_types=[
            pltpu.VMEM((_ISTEPS, 1, _CHUNK), jnp.int32),
            pltpu.VMEM((_ISTEPS, 1, _CHUNK), jnp.int32),
            pltpu.VMEM(g_shape, _F32), pltpu.VMEM(g_shape, _F32),
            pltpu.VMEM(g_shape, _F32), pltpu.VMEM(g_shape, _F32),
            pltpu.VMEM(v_shape, _F32), pltpu.VMEM(v_shape, _F32),
            pltpu.VMEM(v_shape, _F32), pltpu.VMEM(v_shape, _F32),
            pltpu.SemaphoreType.DMA, pltpu.SemaphoreType.DMA,
            pltpu.SemaphoreType.DMA, pltpu.SemaphoreType.DMA,
            pltpu.SemaphoreType.DMA, pltpu.SemaphoreType.DMA,
        ],
    )
    def normk(dis_hbm, row_hbm, col_hbm, ea_hbm, out_hbm, rsup, csup,
              ga0, ga1, gb0, gb1, ev0, ev1, ov0, ov1,
              gsem0, gsem1, esem0, esem1, osem0, osem1):
        c = lax.axis_index("c")
        s = lax.axis_index("s")
        wid = c * _NS + s
        sbase = wid * steps

        def wait2_gather(sem):
            pltpu.make_async_copy(dis_hbm.at[rsup.at[0, 0]], ga0, sem).wait()
            pltpu.make_async_copy(dis_hbm.at[rsup.at[0, 0]], ga0, sem).wait()

        def wait_small(sem):
            pltpu.make_async_copy(ea_hbm.at[0], ev0, sem).wait()

        def compute(ga, gb, ev, ov):
            def srow(r, _):
                w = ga[r, pl.ds(0, 16)] * gb[r, pl.ds(0, 16)]
                ix = pl.ds(16 * (r % 8), 16)
                ov[r // 8, ix] = w * ev[r // 8, ix]
                return 0
            lax.fori_loop(0, _CHUNK, srow, 0, unroll=4)

        for sup in range(nsup):
            off = sbase + sup * _ISTEPS
            pltpu.sync_copy(row_hbm.at[pl.ds(off, _ISTEPS)], rsup)
            pltpu.sync_copy(col_hbm.at[pl.ds(off, _ISTEPS)], csup)
            pltpu.async_copy(dis_hbm.at[rsup.at[0, 0]], ga0, gsem0)
            pltpu.async_copy(dis_hbm.at[csup.at[0, 0]], gb0, gsem0)
            pltpu.async_copy(ea_hbm.at[off], ev0, esem0)

            def dstep(j2, _):
                j = 2 * j2
                pltpu.async_copy(dis_hbm.at[rsup.at[j + 1, 0]], ga1, gsem1)
                pltpu.async_copy(dis_hbm.at[csup.at[j + 1, 0]], gb1, gsem1)
                pltpu.async_copy(ea_hbm.at[off + j + 1], ev1, esem1)
                @pl.when(j2 > 0)
                def _():
                    wait_small(osem0)
                wait2_gather(gsem0)
                wait_small(esem0)
                compute(ga0, gb0, ev0, ov0)
                pltpu.async_copy(ov0, out_hbm.at[off + j], osem0)
                pltpu.async_copy(dis_hbm.at[rsup.at[j + 2, 0]], ga0, gsem0)
                pltpu.async_copy(dis_hbm.at[csup.at[j + 2, 0]], gb0, gsem0)
                pltpu.async_copy(ea_hbm.at[off + j + 2], ev0, esem0)
                @pl.when(j2 > 0)
                def _():
                    wait_small(osem1)
                wait2_gather(gsem1)
                wait_small(esem1)
                compute(ga1, gb1, ev1, ov1)
                pltpu.async_copy(ov1, out_hbm.at[off + j + 1], osem1)
                return 0

            lax.fori_loop(0, (_ISTEPS - 1) // 2, dstep, 0)
            wait_small(osem0)
            wait2_gather(gsem0)
            wait_small(esem0)
            compute(ga0, gb0, ev0, ov0)
            pltpu.async_copy(ov0, out_hbm.at[off + _ISTEPS - 1], osem0)
            wait_small(osem0)
            wait_small(osem1)

    return normk


def _gelu(h):
    return 0.5 * h * (1.0 + lax.erf(h * (2.0 ** -0.5)))


@functools.cache
def _add2_tc(n, d):
    def body(p_ref, o_ref):
        o_ref[...] = p_ref[0] + p_ref[1]

    return pl.pallas_call(body, out_shape=jax.ShapeDtypeStruct((n, d), _F32))


@functools.cache
def _dis_tc(n):
    def body(deg_ref, o_ref):
        deg = deg_ref[0] + deg_ref[1]
        o_ref[...] = jnp.where(deg > 0, lax.rsqrt(deg), 0.0)

    return pl.pallas_call(body, out_shape=jax.ShapeDtypeStruct((n, 128), _F32))


@functools.cache
def _mlpup_tc(n, ins, emb):
    def body(x_ref, w1_ref, b1_ref, w2_ref, b2_ref, o_ref):
        h = jnp.dot(x_ref[...], w1_ref[...], preferred_element_type=_F32)
        h = _gelu(h + b1_ref[...])
        o_ref[...] = (jnp.dot(h, w2_ref[...], preferred_element_type=_F32)
                      + b2_ref[...])

    return pl.pallas_call(body, out_shape=jax.ShapeDtypeStruct((n, emb), _F32))


@functools.cache
def _combine_tc(n, dp, do, act):
    def body(m0_ref, m1_ref, m2_ref, p3_ref, w_ref, b_ref, o_ref):
        acc = jnp.dot(m0_ref[...], w_ref[0], preferred_element_type=_F32)
        acc += jnp.dot(m1_ref[...], w_ref[1], preferred_element_type=_F32)
        acc += jnp.dot(m2_ref[...], w_ref[2], preferred_element_type=_F32)
        acc += jnp.dot(p3_ref[0] + p3_ref[1], w_ref[3],
                       preferred_element_type=_F32)
        acc += b_ref[...]
        if act == "relu":
            o_ref[...] = jnp.maximum(acc, 0.0)
        else:
            shifted = acc - jnp.max(acc, axis=1, keepdims=True)
            lse = jnp.log(jnp.sum(jnp.exp(shifted), axis=1, keepdims=True))
            o_ref[...] = shifted - lse

    return pl.pallas_call(body, out_shape=jax.ShapeDtypeStruct((n, do), _F32))


@functools.cache
def _mlpdown_tc(n, emb):
    def body(z_ref, w1_ref, b1_ref, w2_ref, b2_ref, o_ref):
        h = jnp.dot(z_ref[...], w1_ref[...], preferred_element_type=_F32)
        g = _gelu(h + b1_ref[...])
        o_ref[...] = g * w2_ref[0, 0] + b2_ref[...]

    return pl.pallas_call(body, out_shape=jax.ShapeDtypeStruct((n, 1), _F32))


def _hop(m, row2, col2, nx3, n, e):
    d = m.shape[1]
    part = _spmm_sc(n, e, d)(m, row2, col2, nx3)
    return _add2_tc(n, d)(part)


def kernel(x, pos, edge_attr, params, edge_index):
    n, e = x.shape[0], edge_attr.shape[0]
    row, col = edge_index[0], edge_index[1]
    row2 = row.reshape(-1, 1, _CHUNK)
    col2 = col.reshape(-1, 1, _CHUNK)

    ea16 = jnp.broadcast_to(edge_attr[:, None], (e, 16))
    ea3 = ea16.reshape(-1, _CHUNK // 8, 128)
    ones128 = jnp.ones((n, 128), _F32)
    deg2 = _spmm_sc(n, e, 128)(ones128, row2, col2, ea3)
    dis128 = _dis_tc(n)(deg2)
    nx3 = _nx_sc(n, e)(dis128, row2, col2, ea3)

    pu = params["mlpup"]
    z = _mlpup_tc(n, x.shape[1], pu["w1"].shape[1])(
        x, pu["w1"], pu["b1"], pu["w2"], pu["b2"])

    emb = z.shape[1]
    dp1 = 128
    w1p = jnp.pad(params["conv1"]["w"],
                  ((0, 0), (0, dp1 - params["conv1"]["w"].shape[1]), (0, 0)))
    pad = jnp.zeros((n, dp1 - emb - pos.shape[1]), _F32)

    def tag_layer(m0, w, b, act):
        d = m0.shape[1]
        m1 = _hop(m0, row2, col2, nx3, n, e)
        m2 = _hop(m1, row2, col2, nx3, n, e)
        p3 = _spmm_sc(n, e, d)(m2, row2, col2, nx3)
        return _combine_tc(n, d, w.shape[2], act)(m0, m1, m2, p3, w, b)

    for _ in range(4):
        h0 = jnp.concatenate([z, pos, pad], axis=1)
        h1 = tag_layer(h0, w1p, params["conv1"]["b"], "relu")
        h2 = tag_layer(h1, params["conv2"]["w"], params["conv2"]["b"], "relu")
        z = tag_layer(h2, params["conv3"]["w"], params["conv3"]["b"], "lsm")

    pd = params["mlpdown"]
    out = _mlpdown_tc(n, emb)(z, pd["w1"], pd["b1"], pd["w2"], pd["b2"])
    return (out, z)
